# baseline (speedup 1.0000x reference)
import functools

import jax
import jax.numpy as jnp
from jax import lax
from jax.experimental import pallas as pl
from jax.experimental.pallas import tpu as pltpu
from jax.experimental.pallas import tpu_sc as plsc

BATCH = 4096
SEQ = 50
D = 32
H = 16
B = BATCH * SEQ

_info = plsc.get_sparse_core_info()
NC, NS = _info.num_cores, _info.num_subcores
NW = NC * NS
B_PER_W = B // NW
CHUNK = 400
NCHUNKS = B_PER_W // CHUNK
NBUF = 2


def _sc_embed(u_hbm, i_hbm, c_hbm, halves):
    mesh = plsc.VectorSubcoreMesh(core_axis_name="c", subcore_axis_name="s")

    @functools.partial(
        pl.kernel,
        mesh=mesh,
        out_type=jax.ShapeDtypeStruct((6 * B, H), jnp.float32),
        compiler_params=pltpu.CompilerParams(use_tc_tiling_on_sc=False),
        scratch_types=[
            pltpu.VMEM((NCHUNKS, CHUNK), jnp.int32),
            pltpu.VMEM((NCHUNKS, CHUNK), jnp.int32),
            pltpu.VMEM((NCHUNKS, CHUNK), jnp.int32),
            [pltpu.VMEM((6, CHUNK), jnp.int32)] * NBUF,
            [pltpu.VMEM((CHUNK, H), jnp.float32)] * (6 * NBUF),
            pltpu.SemaphoreType.DMA,
            [pltpu.SemaphoreType.DMA] * NBUF,
            [pltpu.SemaphoreType.DMA] * NBUF,
        ],
    )
    def k(u, i, c, t0, t1, t2, t3, t4, t5, out,
          idx_u, idx_i, idx_c, oidxs, rbufs, sem0, gsems, ssems):
        tabs = (t0, t1, t2, t3, t4, t5)
        wid = lax.axis_index("s") * NC + lax.axis_index("c")
        base_w = wid * B_PER_W

        c1 = pltpu.async_copy(u.at[pl.ds(wid * NCHUNKS, NCHUNKS)], idx_u, sem0)
        c2 = pltpu.async_copy(i.at[pl.ds(wid * NCHUNKS, NCHUNKS)], idx_i, sem0)
        c3 = pltpu.async_copy(c.at[pl.ds(wid * NCHUNKS, NCHUNKS)], idx_c, sem0)
        c1.wait()
        c2.wait()
        c3.wait()

        idxs = (idx_u, idx_u, idx_i, idx_i, idx_c, idx_c)

        def fill_oidx(b, j):
            oidx = oidxs[b]

            def body(m, carry):
                v = 6 * (lax.broadcasted_iota(jnp.int32, (16,), 0)
                         + base_w + j * CHUNK + m * 16)
                for r in range(6):
                    oidx[r, pl.ds(m * 16, 16)] = v + r
                return carry

            lax.fori_loop(0, CHUNK // 16, body, 0)

        def gathers(j, b):
            return [
                pltpu.async_copy(tabs[r].at[idxs[r].at[j]], rbufs[6 * b + r],
                                 gsems[b])
                for r in range(6)
            ]

        def scatters(j, b):
            return [
                pltpu.async_copy(rbufs[6 * b + r], out.at[oidxs[b].at[r]],
                                 ssems[b])
                for r in range(6)
            ]

        g = [None] * NBUF
        s = [None] * NBUF
        for j in range(NCHUNKS):
            b = j % NBUF
            if s[b] is not None:
                for d in s[b]:
                    d.wait()
            fill_oidx(b, j)
            g[b] = gathers(j, b)
            pb = (j - 1) % NBUF
            if j >= 1:
                for d in g[pb]:
                    d.wait()
                s[pb] = scatters(j - 1, pb)
        lb = (NCHUNKS - 1) % NBUF
        for d in g[lb]:
            d.wait()
        s[lb] = scatters(NCHUNKS - 1, lb)
        for b in range(NBUF):
            for d in s[b]:
                d.wait()

    return k(u_hbm, i_hbm, c_hbm, *halves)


def kernel(user_id, item_id, category, W_user_id, W_item_id, W_category):
    u = user_id.reshape(NW * NCHUNKS, CHUNK).astype(jnp.int32)
    i = item_id.reshape(NW * NCHUNKS, CHUNK).astype(jnp.int32)
    c = category.reshape(NW * NCHUNKS, CHUNK).astype(jnp.int32)
    halves = [W[:, r * H:(r + 1) * H]
              for W in (W_user_id, W_item_id, W_category) for r in range(2)]
    out = _sc_embed(u, i, c, halves)
    return out.reshape(BATCH, SEQ, 3 * D)


# speedup vs baseline: 2.0567x; 2.0567x over previous
import functools

import jax
import jax.numpy as jnp
from jax import lax
from jax.experimental import pallas as pl
from jax.experimental.pallas import tpu as pltpu
from jax.experimental.pallas import tpu_sc as plsc

BATCH = 4096
SEQ = 50
D = 32
B = BATCH * SEQ

_info = plsc.get_sparse_core_info()
NC, NS = _info.num_cores, _info.num_subcores
NW = NC * NS
B_PER_W = B // NW
CHUNK = 400
NCHUNKS = B_PER_W // CHUNK
NBUF = 2


def _sc_embed(u_hbm, i_hbm, c_hbm, wu_hbm, wi_hbm, wc_hbm):
    mesh = plsc.VectorSubcoreMesh(core_axis_name="c", subcore_axis_name="s")

    @functools.partial(
        pl.kernel,
        mesh=mesh,
        out_type=jax.ShapeDtypeStruct((3 * B, D), jnp.float32),
        compiler_params=pltpu.CompilerParams(use_tc_tiling_on_sc=False),
        scratch_types=[
            pltpu.VMEM((NCHUNKS, CHUNK), jnp.int32),
            pltpu.VMEM((NCHUNKS, CHUNK), jnp.int32),
            pltpu.VMEM((NCHUNKS, CHUNK), jnp.int32),
            pltpu.VMEM((NCHUNKS, CHUNK), jnp.int32),
            pltpu.VMEM((NCHUNKS, CHUNK), jnp.int32),
            pltpu.VMEM((NCHUNKS, CHUNK), jnp.int32),
            [pltpu.VMEM((CHUNK, D), jnp.float32)] * (3 * NBUF),
            pltpu.SemaphoreType.DMA,
            [pltpu.SemaphoreType.DMA] * NBUF,
            [pltpu.SemaphoreType.DMA] * NBUF,
        ],
    )
    def k(u, i, c, wu, wi, wc, out,
          idx_u, idx_i, idx_c, oidx_u, oidx_i, oidx_c,
          rbufs, sem0, gsems, ssems):
        wid = lax.axis_index("s") * NC + lax.axis_index("c")
        base_w = wid * B_PER_W

        c1 = pltpu.async_copy(u.at[pl.ds(wid * NCHUNKS, NCHUNKS)], idx_u, sem0)
        c2 = pltpu.async_copy(i.at[pl.ds(wid * NCHUNKS, NCHUNKS)], idx_i, sem0)
        c3 = pltpu.async_copy(c.at[pl.ds(wid * NCHUNKS, NCHUNKS)], idx_c, sem0)

        def fill_chunk(j, _):
            def fill_vec(m, _):
                v = 3 * (lax.broadcasted_iota(jnp.int32, (16,), 0)
                         + base_w + j * CHUNK + m * 16)
                oidx_u[j, pl.ds(m * 16, 16)] = v
                oidx_i[j, pl.ds(m * 16, 16)] = v + 1
                oidx_c[j, pl.ds(m * 16, 16)] = v + 2
                return 0
            return lax.fori_loop(0, CHUNK // 16, fill_vec, 0)

        lax.fori_loop(0, NCHUNKS, fill_chunk, 0)
        c1.wait()
        c2.wait()
        c3.wait()

        def gathers(j, b):
            return [
                pltpu.async_copy(wu.at[idx_u.at[j]], rbufs[3 * b + 0], gsems[b]),
                pltpu.async_copy(wi.at[idx_i.at[j]], rbufs[3 * b + 1], gsems[b]),
                pltpu.async_copy(wc.at[idx_c.at[j]], rbufs[3 * b + 2], gsems[b]),
            ]

        def scatters(j, b):
            return [
                pltpu.async_copy(rbufs[3 * b + 0], out.at[oidx_u.at[j]], ssems[b]),
                pltpu.async_copy(rbufs[3 * b + 1], out.at[oidx_i.at[j]], ssems[b]),
                pltpu.async_copy(rbufs[3 * b + 2], out.at[oidx_c.at[j]], ssems[b]),
            ]

        g = [None] * NBUF
        s = [None] * NBUF
        for j in range(NCHUNKS):
            b = j % NBUF
            if s[b] is not None:
                for d in s[b]:
                    d.wait()
            g[b] = gathers(j, b)
            pb = (j - 1) % NBUF
            if j >= 1:
                for d in g[pb]:
                    d.wait()
                s[pb] = scatters(j - 1, pb)
        lb = (NCHUNKS - 1) % NBUF
        for d in g[lb]:
            d.wait()
        s[lb] = scatters(NCHUNKS - 1, lb)
        for b in range(NBUF):
            for d in s[b]:
                d.wait()

    return k(u_hbm, i_hbm, c_hbm, wu_hbm, wi_hbm, wc_hbm)


def kernel(user_id, item_id, category, W_user_id, W_item_id, W_category):
    u = user_id.reshape(NW * NCHUNKS, CHUNK).astype(jnp.int32)
    i = item_id.reshape(NW * NCHUNKS, CHUNK).astype(jnp.int32)
    c = category.reshape(NW * NCHUNKS, CHUNK).astype(jnp.int32)
    out = _sc_embed(u, i, c, W_user_id, W_item_id, W_category)
    return out.reshape(BATCH, SEQ, 3 * D)
